# baseline (speedup 1.0000x reference)
import functools
import math

import jax
import jax.numpy as jnp
from jax import lax
from jax.experimental import pallas as pl
from jax.experimental.pallas import tpu as pltpu
from jax.experimental.pallas import tpu_sc as plsc

_NC = 2
_NS = 16
_L = 16


def _router_body(x_ref, wr_ref, br_ref, p_ref, xb_ref):
    x = x_ref[...]
    logits = jnp.dot(x, wr_ref[...], preferred_element_type=jnp.float32)
    logits = logits + br_ref[...]
    T, E = logits.shape
    m = jnp.max(logits, axis=-1, keepdims=True)
    ex = jnp.exp(logits - m)
    p = ex / jnp.sum(ex, axis=-1, keepdims=True)
    lane = lax.broadcasted_iota(jnp.int32, (T, E), 1)
    m1 = jnp.max(p, axis=-1, keepdims=True)
    a1 = jnp.min(jnp.where(p == m1, lane, E), axis=-1, keepdims=True)
    p2 = jnp.where(lane == a1, -1.0, p)
    m2 = jnp.max(p2, axis=-1, keepdims=True)
    a2 = jnp.min(jnp.where(p2 == m2, lane, E), axis=-1, keepdims=True)
    s = m1 + m2
    p_ref[...] = jnp.concatenate(
        [jnp.where(lane == a1, m1 / s, 0.0),
         jnp.where(lane == a2, m2 / s, 0.0)], axis=-1)
    xb_ref[...] = x.astype(jnp.bfloat16)


def _router(xf, Wr, br2):
    T, D = xf.shape
    E = Wr.shape[1]
    return pl.pallas_call(
        _router_body,
        out_shape=(jax.ShapeDtypeStruct((T, 2 * E), jnp.float32),
                   jax.ShapeDtypeStruct((T, D), jnp.bfloat16)),
    )(xf, Wr, br2)


def _make_dispatch(T, D, E, CAP):
    NW = _NC * _NS
    SPW = (E * CAP) // NW
    TPW = T // NW
    assert (E * CAP) % NW == 0 and SPW % 2 == 0 and T % NW == 0
    HALF = SPW // 2
    CP = CAP + _L
    ZEROSLOT = E * CAP
    mesh = plsc.VectorSubcoreMesh(core_axis_name="c", subcore_axis_name="s",
                                  num_cores=_NC, num_subcores=_NS)

    @functools.partial(
        pl.kernel,
        out_type=(
            jax.ShapeDtypeStruct((E * CAP, D // 2), jnp.int32),
            jax.ShapeDtypeStruct((E, CAP), jnp.float32),
            jax.ShapeDtypeStruct((T,), jnp.int32),
            jax.ShapeDtypeStruct((T,), jnp.int32),
        ),
        mesh=mesh,
        compiler_params=pltpu.CompilerParams(needs_layout_passes=False),
        scratch_types=[
            pltpu.VMEM((T,), jnp.float32),
            pltpu.VMEM((T,), jnp.float32),
            pltpu.VMEM((CP,), jnp.int32),
            pltpu.VMEM((CP,), jnp.float32),
            pltpu.VMEM((T,), jnp.int32),
            pltpu.VMEM((T,), jnp.int32),
            pltpu.VMEM((HALF,), jnp.int32),
            pltpu.VMEM((HALF,), jnp.int32),
            pltpu.VMEM((SPW, D // 2), jnp.int32),
            pltpu.VMEM((2 * E, TPW), jnp.int32),
            pltpu.VMEM((TPW,), jnp.int32),
            pltpu.VMEM((TPW,), jnp.int32),
            pltpu.VMEM_SHARED((E * CAP,), jnp.int32),
            pltpu.VMEM_SHARED((2 * E * T,), jnp.int32),
            pltpu.SemaphoreType.DMA,
            pltpu.SemaphoreType.DMA,
        ],
    )
    def dispatch(pt_hbm, xb_hbm, xg_hbm, w_hbm, inv1_hbm, inv2_hbm,
                 p1buf, p2buf, ibuf, wbuf, inv1buf, inv2buf,
                 gidx0, gidx1, rowbuf, mtmp, mout1, mout2, sidx, sinv,
                 semA, semB):
        c = lax.axis_index("c")
        s = lax.axis_index("s")

        @pl.when(s < E)
        def _phase_a():
            cp1 = pltpu.async_copy(pt_hbm.at[s], p1buf, semA)
            cp2 = pltpu.async_copy(pt_hbm.at[E + s], p2buf, semB)
            cp1.wait()
            cp2.wait()

            def zb(i, carry):
                ibuf[pl.ds(i * _L, _L)] = jnp.zeros((_L,), jnp.int32)
                wbuf[pl.ds(i * _L, _L)] = jnp.zeros((_L,), jnp.float32)
                return carry

            lax.fori_loop(0, CAP // _L, zb, 0)
            slot0 = s * CAP

            def step(t0, ptr):
                pv1 = p1buf[pl.ds(t0 * _L, _L)]
                pv2 = p2buf[pl.ds(t0 * _L, _L)]
                m1 = pv1 > 0.0
                m2 = pv2 > 0.0
                msk = m1 | m2
                mi = msk.astype(jnp.int32)
                tok = lax.iota(jnp.int32, _L) + t0 * _L
                pos = ptr + plsc.cumsum(mi) - mi
                posc = jnp.minimum(pos, CAP)
                plsc.store_scatter(ibuf, [posc], tok, mask=msk)
                plsc.store_scatter(wbuf, [posc], pv1 + pv2, mask=msk)
                ok = msk & (pos < CAP)
                slot = jnp.where(ok, slot0 + pos, ZEROSLOT)
                inv1buf[pl.ds(t0 * _L, _L)] = jnp.where(m1, slot, ZEROSLOT)
                inv2buf[pl.ds(t0 * _L, _L)] = jnp.where(m2, slot, ZEROSLOT)
                return ptr + jnp.sum(mi)

            lax.fori_loop(0, T // _L, step, jnp.int32(0))
            cps = [
                pltpu.async_copy(ibuf.at[pl.ds(0, CAP)],
                                 sidx.at[pl.ds(s * CAP, CAP)], semA),
                pltpu.async_copy(inv1buf, sinv.at[pl.ds(s * T, T)], semA),
                pltpu.async_copy(inv2buf, sinv.at[pl.ds((E + s) * T, T)], semA),
            ]

            @pl.when(c == 0)
            def _():
                pltpu.async_copy(wbuf.at[pl.ds(0, CAP)], w_hbm.at[s],
                                 semB).wait()

            for cp in cps:
                cp.wait()

        plsc.subcore_barrier()

        wid = s * _NC + c
        t0 = wid * TPW
        mcps = [pltpu.async_copy(sinv.at[pl.ds(e * T + t0, TPW)],
                                 mtmp.at[e], semA) for e in range(2 * E)]
        for cp in mcps:
            cp.wait()

        def mmerge(k, carry):
            v1 = mtmp[0, pl.ds(k * _L, _L)]
            v2 = mtmp[E, pl.ds(k * _L, _L)]
            for e in range(1, E):
                v1 = jnp.minimum(v1, mtmp[e, pl.ds(k * _L, _L)])
                v2 = jnp.minimum(v2, mtmp[E + e, pl.ds(k * _L, _L)])
            mout1[pl.ds(k * _L, _L)] = v1
            mout2[pl.ds(k * _L, _L)] = v2
            return carry

        lax.fori_loop(0, TPW // _L, mmerge, 0, unroll=2)
        mo1 = pltpu.async_copy(mout1, inv1_hbm.at[pl.ds(t0, TPW)], semB)
        mo2 = pltpu.async_copy(mout2, inv2_hbm.at[pl.ds(t0, TPW)], semB)

        wpe = CAP // SPW
        e = wid // wpe
        o = (wid % wpe) * SPW
        base = e * CAP + o
        gi0 = pltpu.async_copy(sidx.at[pl.ds(base, HALF)], gidx0, semA)
        gi1 = pltpu.async_copy(sidx.at[pl.ds(base + HALF, HALF)], gidx1, semA)
        gi0.wait()
        gi1.wait()
        g0 = pltpu.async_copy(xb_hbm.at[gidx0], rowbuf.at[pl.ds(0, HALF)], semA)
        g1 = pltpu.async_copy(xb_hbm.at[gidx1], rowbuf.at[pl.ds(HALF, HALF)],
                              semA)
        g0.wait()
        g1.wait()
        pltpu.async_copy(rowbuf, xg_hbm.at[pl.ds(base, SPW)], semA).wait()
        mo1.wait()
        mo2.wait()

    return dispatch


def _make_ffn(E, CAP, D, FF, FBLK):
    NF = FF // FBLK
    cl = lambda e: jnp.minimum(e, E - 1)

    def body(xg_ref, w1_ref, b1_ref, w2_ref, b2_ref, wc_ref, out_ref):
        f = pl.program_id(1)
        xb = xg_ref[...].astype(jnp.bfloat16)
        w1 = w1_ref[0].astype(jnp.bfloat16)
        h = jnp.dot(xb, w1, preferred_element_type=jnp.float32) + b1_ref[0]
        h = jnp.maximum(h, 0.0).astype(jnp.bfloat16)
        w2 = w2_ref[0].astype(jnp.bfloat16)
        part = jnp.dot(h, w2, preferred_element_type=jnp.float32)

        @pl.when(f == 0)
        def _():
            out_ref[...] = part + b2_ref[0]

        @pl.when(f != 0)
        def _():
            out_ref[...] += part

        @pl.when(f == NF - 1)
        def _():
            wv = wc_ref[0]
            out_ref[...] *= jnp.concatenate([wv] * (D // 128), axis=1)

    return pl.pallas_call(
        body,
        grid=(E + 1, NF),
        in_specs=[
            pl.BlockSpec((CAP, D), lambda e, f: (cl(e), 0)),
            pl.BlockSpec((1, D, FBLK), lambda e, f: (cl(e), 0, f)),
            pl.BlockSpec((1, 1, FBLK), lambda e, f: (cl(e), 0, f)),
            pl.BlockSpec((1, FBLK, D), lambda e, f: (cl(e), f, 0)),
            pl.BlockSpec((1, 1, D), lambda e, f: (cl(e), 0, 0)),
            pl.BlockSpec((1, CAP, 128), lambda e, f: (e, 0, 0)),
        ],
        out_specs=pl.BlockSpec((CAP, D), lambda e, f: (e, 0)),
        out_shape=jax.ShapeDtypeStruct(((E + 1) * CAP, D), jnp.float32),
        compiler_params=pltpu.CompilerParams(
            dimension_semantics=("arbitrary", "arbitrary")),
    )


def _make_combine(T, D, E, CAP):
    NW = _NC * _NS
    TPW = T // NW
    assert T % NW == 0
    mesh = plsc.VectorSubcoreMesh(core_axis_name="c", subcore_axis_name="s",
                                  num_cores=_NC, num_subcores=_NS)

    @functools.partial(
        pl.kernel,
        out_type=jax.ShapeDtypeStruct((T, D), jnp.float32),
        mesh=mesh,
        compiler_params=pltpu.CompilerParams(needs_layout_passes=False),
        scratch_types=[
            pltpu.VMEM((TPW,), jnp.int32),
            pltpu.VMEM((TPW,), jnp.int32),
            pltpu.VMEM((TPW, D), jnp.float32),
            pltpu.VMEM((TPW, D), jnp.float32),
            pltpu.SemaphoreType.DMA,
            pltpu.SemaphoreType.DMA,
        ],
    )
    def combine(yw_hbm, inv1_hbm, inv2_hbm, out_hbm, g1, g2, buf1, buf2,
                sem1, sem2):
        c = lax.axis_index("c")
        s = lax.axis_index("s")
        wid = s * _NC + c
        t0 = wid * TPW
        i1 = pltpu.async_copy(inv1_hbm.at[pl.ds(t0, TPW)], g1, sem1)
        i2 = pltpu.async_copy(inv2_hbm.at[pl.ds(t0, TPW)], g2, sem2)
        i1.wait()
        i2.wait()
        cp1 = pltpu.async_copy(yw_hbm.at[g1], buf1, sem1)
        cp2 = pltpu.async_copy(yw_hbm.at[g2], buf2, sem2)
        cp1.wait()
        cp2.wait()

        def acc(i, carry):
            r = i // (D // _L)
            l = (i % (D // _L)) * _L
            buf1[r, pl.ds(l, _L)] = (buf1[r, pl.ds(l, _L)]
                                     + buf2[r, pl.ds(l, _L)])
            return carry

        lax.fori_loop(0, TPW * (D // _L), acc, 0, unroll=8)
        pltpu.sync_copy(buf1, out_hbm.at[pl.ds(t0, TPW)])

    return combine


def kernel(x, Wr, br, W1, b1, W2, b2):
    Bv, Sv, D = x.shape
    T = Bv * Sv
    E = Wr.shape[1]
    FF = W1.shape[2]
    CAP = max(int(math.ceil(1.25 * T * 2 / E)), 4)

    xf = x.reshape(T, D)
    P, xb = _router(xf, Wr, br.reshape(1, E))
    PT = P.T

    dispatch = _make_dispatch(T, D, E, CAP)
    xb32 = lax.bitcast_convert_type(xb.reshape(T, D // 2, 2), jnp.int32)
    xg32, w, inv1, inv2 = dispatch(PT, xb32)
    xg = lax.bitcast_convert_type(xg32, jnp.bfloat16).reshape(E * CAP, D)

    wc = jnp.broadcast_to(
        jnp.concatenate([w, jnp.zeros((1, CAP), jnp.float32)])[..., None],
        (E + 1, CAP, 128))
    ffn = _make_ffn(E, CAP, D, FF, 768)
    yw = ffn(xg, W1, b1.reshape(E, 1, FF), W2, b2.reshape(E, 1, D), wc)

    combine = _make_combine(T, D, E, CAP)
    out = combine(yw, inv1, inv2)
    return out.reshape(Bv, Sv, D)


# speedup vs baseline: 1.6864x; 1.6864x over previous
import functools
import math

import jax
import jax.numpy as jnp
from jax import lax
from jax.experimental import pallas as pl
from jax.experimental.pallas import tpu as pltpu
from jax.experimental.pallas import tpu_sc as plsc

_NC = 2
_NS = 16
_L = 16


def _router_body(x_ref, wr_ref, br_ref, p_ref):
    x = x_ref[...]
    logits = jnp.dot(x, wr_ref[...], preferred_element_type=jnp.float32)
    logits = logits + br_ref[...]
    T, E = logits.shape
    m = jnp.max(logits, axis=-1, keepdims=True)
    ex = jnp.exp(logits - m)
    p = ex / jnp.sum(ex, axis=-1, keepdims=True)
    lane = lax.broadcasted_iota(jnp.int32, (T, E), 1)
    m1 = jnp.max(p, axis=-1, keepdims=True)
    a1 = jnp.min(jnp.where(p == m1, lane, E), axis=-1, keepdims=True)
    p2 = jnp.where(lane == a1, -1.0, p)
    m2 = jnp.max(p2, axis=-1, keepdims=True)
    a2 = jnp.min(jnp.where(p2 == m2, lane, E), axis=-1, keepdims=True)
    s = m1 + m2
    p_ref[...] = jnp.where(lane == a1, m1 / s,
                           jnp.where(lane == a2, -(m2 / s), 0.0))


def _router(xf, Wr, br2):
    T, D = xf.shape
    E = Wr.shape[1]
    return pl.pallas_call(
        _router_body,
        out_shape=jax.ShapeDtypeStruct((T, E), jnp.float32),
    )(xf, Wr, br2)


def _make_dispatch(T, D, E, CAP):
    NW = _NC * _NS
    SPW = (E * CAP) // NW
    assert (E * CAP) % NW == 0 and SPW % 2 == 0 and T % NW == 0
    HALF = SPW // 2
    CP = CAP + _L
    ZEROSLOT = E * CAP
    mesh = plsc.VectorSubcoreMesh(core_axis_name="c", subcore_axis_name="s",
                                  num_cores=_NC, num_subcores=_NS)

    @functools.partial(
        pl.kernel,
        out_type=(
            jax.ShapeDtypeStruct((E * CAP, D), jnp.float32),
            jax.ShapeDtypeStruct((E, CAP), jnp.float32),
            jax.ShapeDtypeStruct((2 * E * T,), jnp.int32),
            jax.ShapeDtypeStruct((2 * E * CAP,), jnp.int32),
        ),
        mesh=mesh,
        compiler_params=pltpu.CompilerParams(needs_layout_passes=False),
        scratch_types=[
            pltpu.VMEM((T,), jnp.float32),
            pltpu.VMEM((CP,), jnp.int32),
            pltpu.VMEM((CP,), jnp.float32),
            pltpu.VMEM((T,), jnp.int32),
            pltpu.VMEM((T,), jnp.int32),
            pltpu.VMEM((HALF,), jnp.int32),
            pltpu.VMEM((HALF,), jnp.int32),
            pltpu.VMEM((HALF, D), jnp.float32),
            pltpu.VMEM((HALF, D), jnp.float32),
            pltpu.SemaphoreType.DMA,
            pltpu.SemaphoreType.DMA,
        ],
    )
    def dispatch(pt_hbm, x_hbm, xg_hbm, w_hbm, inva_hbm, idxc_hbm,
                 pbuf, ibuf, wbuf, inv1buf, inv2buf,
                 gidx0, gidx1, rowbuf0, rowbuf1, semA, semB):
        c = lax.axis_index("c")
        s = lax.axis_index("s")

        @pl.when(s < E)
        def _phase_a():
            pltpu.async_copy(pt_hbm.at[s], pbuf, semA).wait()

            def zb(i, carry):
                ibuf[pl.ds(i * _L, _L)] = jnp.zeros((_L,), jnp.int32)
                wbuf[pl.ds(i * _L, _L)] = jnp.zeros((_L,), jnp.float32)
                return carry

            lax.fori_loop(0, CAP // _L, zb, 0)
            slot0 = s * CAP

            def step(t0, ptr):
                pv = pbuf[pl.ds(t0 * _L, _L)]
                m1 = pv > 0.0
                m2 = pv < 0.0
                msk = m1 | m2
                mi = msk.astype(jnp.int32)
                tok = lax.iota(jnp.int32, _L) + t0 * _L
                pos = ptr + plsc.cumsum(mi) - mi
                posc = jnp.minimum(pos, CAP)
                plsc.store_scatter(ibuf, [posc], tok, mask=msk)
                plsc.store_scatter(wbuf, [posc], jnp.abs(pv), mask=msk)
                ok = msk & (pos < CAP)
                slot = jnp.where(ok, slot0 + pos, ZEROSLOT)
                inv1buf[pl.ds(t0 * _L, _L)] = jnp.where(m1, slot, ZEROSLOT)
                inv2buf[pl.ds(t0 * _L, _L)] = jnp.where(m2, slot, ZEROSLOT)
                return ptr + jnp.sum(mi)

            lax.fori_loop(0, T // _L, step, jnp.int32(0))
            cp0 = pltpu.async_copy(ibuf.at[pl.ds(0, CAP)],
                                   idxc_hbm.at[pl.ds((c * E + s) * CAP, CAP)],
                                   semA)

            @pl.when(c == 0)
            def _():
                cw = pltpu.async_copy(wbuf.at[pl.ds(0, CAP)],
                                      w_hbm.at[s], semB)
                ci1 = pltpu.async_copy(inv1buf,
                                       inva_hbm.at[pl.ds(s * T, T)], semB)
                ci2 = pltpu.async_copy(inv2buf,
                                       inva_hbm.at[pl.ds((E + s) * T, T)],
                                       semB)
                cw.wait()
                ci1.wait()
                ci2.wait()

            cp0.wait()

        plsc.subcore_barrier()

        wid = s * _NC + c
        wpe = CAP // SPW
        e = wid // wpe
        o = (wid % wpe) * SPW
        base = e * CAP + o
        cbase = (c * E + e) * CAP + o
        gi0 = pltpu.async_copy(idxc_hbm.at[pl.ds(cbase, HALF)], gidx0, semA)
        gi1 = pltpu.async_copy(idxc_hbm.at[pl.ds(cbase + HALF, HALF)],
                               gidx1, semB)
        gi0.wait()
        g0 = pltpu.async_copy(x_hbm.at[gidx0], rowbuf0, semA)
        gi1.wait()
        g1 = pltpu.async_copy(x_hbm.at[gidx1], rowbuf1, semB)
        g0.wait()
        w0 = pltpu.async_copy(rowbuf0, xg_hbm.at[pl.ds(base, HALF)], semA)
        g1.wait()
        w1 = pltpu.async_copy(rowbuf1, xg_hbm.at[pl.ds(base + HALF, HALF)],
                              semB)
        w0.wait()
        w1.wait()

    return dispatch


def _make_ffn(E, CAP, D, FF, FBLK):
    NF = FF // FBLK
    cl = lambda e: jnp.minimum(e, E - 1)

    def body(xg_ref, w1_ref, b1_ref, w2_ref, b2_ref, wc_ref, out_ref):
        f = pl.program_id(1)
        h = jnp.dot(xg_ref[...], w1_ref[0],
                    preferred_element_type=jnp.float32) + b1_ref[0]
        h = jnp.maximum(h, 0.0)
        part = jnp.dot(h, w2_ref[0], preferred_element_type=jnp.float32)

        @pl.when(f == 0)
        def _():
            out_ref[...] = part + b2_ref[0]

        @pl.when(f != 0)
        def _():
            out_ref[...] += part

        @pl.when(f == NF - 1)
        def _():
            wv = wc_ref[0]
            out_ref[...] *= jnp.concatenate([wv] * (D // 128), axis=1)

    return pl.pallas_call(
        body,
        grid=(E + 1, NF),
        in_specs=[
            pl.BlockSpec((CAP, D), lambda e, f: (cl(e), 0)),
            pl.BlockSpec((1, D, FBLK), lambda e, f: (cl(e), 0, f)),
            pl.BlockSpec((1, 1, FBLK), lambda e, f: (cl(e), 0, f)),
            pl.BlockSpec((1, FBLK, D), lambda e, f: (cl(e), f, 0)),
            pl.BlockSpec((1, 1, D), lambda e, f: (cl(e), 0, 0)),
            pl.BlockSpec((1, CAP, 128), lambda e, f: (e, 0, 0)),
        ],
        out_specs=pl.BlockSpec((CAP, D), lambda e, f: (e, 0)),
        out_shape=jax.ShapeDtypeStruct(((E + 1) * CAP, D), jnp.float32),
        compiler_params=pltpu.CompilerParams(
            dimension_semantics=("arbitrary", "arbitrary")),
    )


def _make_combine(T, D, E, CAP):
    NW = _NC * _NS
    TPW = T // NW
    assert T % NW == 0 and TPW % _L == 0
    mesh = plsc.VectorSubcoreMesh(core_axis_name="c", subcore_axis_name="s",
                                  num_cores=_NC, num_subcores=_NS)

    @functools.partial(
        pl.kernel,
        out_type=jax.ShapeDtypeStruct((T, D), jnp.float32),
        mesh=mesh,
        compiler_params=pltpu.CompilerParams(needs_layout_passes=False),
        scratch_types=[
            pltpu.VMEM((2 * E, TPW), jnp.int32),
            pltpu.VMEM((TPW,), jnp.int32),
            pltpu.VMEM((TPW,), jnp.int32),
            pltpu.VMEM((TPW, D), jnp.float32),
            pltpu.VMEM((TPW, D), jnp.float32),
            pltpu.SemaphoreType.DMA,
            pltpu.SemaphoreType.DMA,
        ],
    )
    def combine(yw_hbm, inva_hbm, out_hbm, mtmp, g1, g2, buf1, buf2,
                sem1, sem2):
        c = lax.axis_index("c")
        s = lax.axis_index("s")
        wid = s * _NC + c
        t0 = wid * TPW
        mcps = [pltpu.async_copy(inva_hbm.at[pl.ds(e * T + t0, TPW)],
                                 mtmp.at[e], sem1) for e in range(2 * E)]
        for cp in mcps:
            cp.wait()

        def mmerge(k, carry):
            v1 = mtmp[0, pl.ds(k * _L, _L)]
            v2 = mtmp[E, pl.ds(k * _L, _L)]
            for e in range(1, E):
                v1 = jnp.minimum(v1, mtmp[e, pl.ds(k * _L, _L)])
                v2 = jnp.minimum(v2, mtmp[E + e, pl.ds(k * _L, _L)])
            g1[pl.ds(k * _L, _L)] = v1
            g2[pl.ds(k * _L, _L)] = v2
            return carry

        lax.fori_loop(0, TPW // _L, mmerge, 0, unroll=4)
        cp1 = pltpu.async_copy(yw_hbm.at[g1], buf1, sem1)
        cp2 = pltpu.async_copy(yw_hbm.at[g2], buf2, sem2)
        cp1.wait()
        cp2.wait()

        def acc(i, carry):
            r = i // (D // _L)
            l = (i % (D // _L)) * _L
            buf1[r, pl.ds(l, _L)] = (buf1[r, pl.ds(l, _L)]
                                     + buf2[r, pl.ds(l, _L)])
            return carry

        lax.fori_loop(0, TPW * (D // _L), acc, 0, unroll=8)
        pltpu.async_copy(buf1, out_hbm.at[pl.ds(t0, TPW)], sem1).wait()

    return combine


def kernel(x, Wr, br, W1, b1, W2, b2):
    Bv, Sv, D = x.shape
    T = Bv * Sv
    E = Wr.shape[1]
    FF = W1.shape[2]
    CAP = max(int(math.ceil(1.25 * T * 2 / E)), 4)

    xf = x.reshape(T, D)
    P = _router(xf, Wr, br.reshape(1, E))
    PT = P.T

    dispatch = _make_dispatch(T, D, E, CAP)
    xg, w, inva, _ = dispatch(PT, xf)

    wc = jnp.broadcast_to(
        jnp.concatenate([w, jnp.zeros((1, CAP), jnp.float32)])[..., None],
        (E + 1, CAP, 128))
    ffn = _make_ffn(E, CAP, D, FF, 3072)
    yw = ffn(xg, W1, b1.reshape(E, 1, FF), W2, b2.reshape(E, 1, D), wc)

    combine = _make_combine(T, D, E, CAP)
    out = combine(yw, inva)
    return out.reshape(Bv, Sv, D)
